# Initial kernel scaffold; baseline (speedup 1.0000x reference)
import jax
import jax.numpy as jnp
from jax.experimental import pallas as pl


def kernel(z):
    raise NotImplementedError("write your pallas kernel here")


# speedup vs baseline: 146.2985x; 146.2985x over previous
import dataclasses
import functools

import jax
import jax.numpy as jnp
from jax import lax
from jax.experimental import pallas as pl
from jax.experimental.pallas import tpu as pltpu
from jax.experimental.pallas import tpu_sc as plsc

_NUM_LAYERS = 8
_NUM_LEVELS = 7
_D = 64
_L = 16
_BLK = 16


def _sc_rfsq(z2, btab):
    rows = z2.shape[0]
    mesh = plsc.VectorSubcoreMesh(core_axis_name="c", subcore_axis_name="s")

    out_type = (
        jax.ShapeDtypeStruct((rows, _D), jnp.float32),
        jax.ShapeDtypeStruct((rows, _D * _NUM_LAYERS), jnp.int32),
    )

    cp = pltpu.CompilerParams()
    if "needs_layout_passes" in pltpu.CompilerParams.__dataclass_fields__:
        cp = dataclasses.replace(cp, needs_layout_passes=False)

    @functools.partial(
        pl.kernel,
        out_type=out_type,
        mesh=mesh,
        compiler_params=cp,
        scratch_types=[
            pltpu.VMEM((_L,), jnp.float32),
            pltpu.VMEM((_D * _L,), jnp.float32),
        ],
    )
    def k(z_hbm, b_hbm, qsum_hbm, codes_hbm, b_v, residT):
        pltpu.sync_copy(b_hbm, b_v)

        def body(z_v, qsum_v, codes_v):
            iota = lax.iota(jnp.int32, _L)
            zero = jnp.zeros((_L,), jnp.float32)

            def tr_body(d, carry):
                s, q = carry
                col = jnp.full((_L,), d, jnp.int32)
                x = plsc.load_gather(z_v, [iota, col])
                residT[pl.ds(d * _L, _L)] = x
                return s + x, q + x * x

            s, q = lax.fori_loop(0, _D, tr_body, (zero, zero))

            for layer in range(_NUM_LAYERS):
                mean = s * (1.0 / _D)
                var = (q - s * mean) * (1.0 / (_D - 1))
                v2 = jnp.maximum(var, 1e-30)
                bits = plsc.bitcast(v2, jnp.int32)
                bits = jnp.int32(0x5F3759DF) - (bits >> 1)
                y = plsc.bitcast(bits, jnp.float32)
                y = y * (1.5 - 0.5 * v2 * y * y)
                y = y * (1.5 - 0.5 * v2 * y * y)
                sd = v2 * y
                sd = 0.5 * (sd + v2 / sd)
                std = sd + 1e-5
                c1 = 3.0 / std
                c2 = 3.5 - mean * c1

                def q_body(d, carry, layer=layer, std=std, mean=mean,
                           c1=c1, c2=c2):
                    s2, q2 = carry
                    x = residT[pl.ds(d * _L, _L)]
                    t = x * c1 + c2
                    t = jnp.minimum(jnp.maximum(t, 0.5), 6.5)
                    idx = t.astype(jnp.int32)
                    col = jnp.full((_L,), d * _NUM_LAYERS + layer, jnp.int32)
                    plsc.store_scatter(codes_v, [iota, col], idx)
                    bq = plsc.load_gather(b_v, [idx])
                    zq = bq * std + mean
                    r = x - zq
                    residT[pl.ds(d * _L, _L)] = r
                    return s2 + r, q2 + r * r

                s, q = lax.fori_loop(0, _D, q_body, (zero, zero))

            def f_body(d, carry):
                col = jnp.full((_L,), d, jnp.int32)
                xz = plsc.load_gather(z_v, [iota, col])
                r = residT[pl.ds(d * _L, _L)]
                plsc.store_scatter(qsum_v, [iota, col], xz - r)
                return carry

            lax.fori_loop(0, _D, f_body, 0)

        pltpu.emit_pipeline(
            body,
            grid=(rows // _BLK,),
            in_specs=[pl.BlockSpec((_BLK, _D), lambda i: (i, 0))],
            out_specs=[
                pl.BlockSpec((_BLK, _D), lambda i: (i, 0)),
                pl.BlockSpec((_BLK, _D * _NUM_LAYERS), lambda i: (i, 0)),
            ],
            core_axis_name=("c", "s"),
            dimension_semantics=(pltpu.PARALLEL,),
        )(z_hbm, qsum_hbm, codes_hbm)

    return k(z2, btab)


def kernel(z):
    b, s, d = z.shape
    z2 = z.reshape(b * s, d)
    bnd = jnp.linspace(-1.0, 1.0, _NUM_LEVELS).astype(jnp.float32)
    btab = jnp.concatenate([bnd, jnp.zeros((_L - _NUM_LEVELS,), jnp.float32)])
    qsum2, codes2 = _sc_rfsq(z2, btab)
    return qsum2.reshape(b, s, d), codes2.reshape(b, s, d, _NUM_LAYERS)
